# baseline (speedup 1.0000x reference)
import functools

import jax
import jax.numpy as jnp
from jax import lax
from jax.experimental import pallas as pl
from jax.experimental.pallas import tpu as pltpu
from jax.experimental.pallas import tpu_sc as plsc

HIDDEN = 64
NUM_TOKENS = 4096 * 200
NUM_WORKERS = 32
PER_WORKER = NUM_TOKENS // NUM_WORKERS
CHUNK = 128
N_CHUNKS = PER_WORKER // CHUNK


@jax.jit
def _embed(idx, table):
    mesh = plsc.VectorSubcoreMesh(core_axis_name="c", subcore_axis_name="s")

    @functools.partial(
        pl.kernel,
        mesh=mesh,
        compiler_params=pltpu.CompilerParams(use_tc_tiling_on_sc=False),
        out_type=jax.ShapeDtypeStruct((NUM_TOKENS, HIDDEN), jnp.float32),
        scratch_types=[
            pltpu.VMEM((N_CHUNKS, CHUNK), jnp.int32),
            pltpu.VMEM((CHUNK, HIDDEN), jnp.float32),
            pltpu.SemaphoreType.DMA,
        ],
    )
    def k(idx_hbm, table_hbm, out_hbm, idx_v, rows_v, gsem):
        wid = lax.axis_index("s") * 2 + lax.axis_index("c")
        base = wid * PER_WORKER
        pltpu.sync_copy(idx_hbm.at[wid], idx_v)

        def body(c, carry):
            pltpu.async_copy(table_hbm.at[idx_v.at[c]], rows_v, gsem).wait()
            pltpu.sync_copy(rows_v, out_hbm.at[pl.ds(base + c * CHUNK, CHUNK)])
            return carry

        lax.fori_loop(0, N_CHUNKS, body, 0)

    return k(idx, table)


def kernel(tokens, embedding_weight):
    b, t = tokens.shape
    idx = tokens.astype(jnp.int32).reshape(NUM_WORKERS, N_CHUNKS, CHUNK)
    out = _embed(idx, embedding_weight)
    return out.reshape(b, t, HIDDEN)


# speedup vs baseline: 1.1173x; 1.1173x over previous
import functools

import jax
import jax.numpy as jnp
from jax import lax
from jax.experimental import pallas as pl
from jax.experimental.pallas import tpu as pltpu
from jax.experimental.pallas import tpu_sc as plsc

HIDDEN = 64
NUM_TOKENS = 4096 * 200
NUM_WORKERS = 32
PER_WORKER = NUM_TOKENS // NUM_WORKERS
CHUNK = 128
N_CHUNKS = PER_WORKER // CHUNK
G = 5
ROWS_G = G * CHUNK
NG = N_CHUNKS // G


@jax.jit
def _embed(idx, table):
    mesh = plsc.VectorSubcoreMesh(core_axis_name="c", subcore_axis_name="s")

    @functools.partial(
        pl.kernel,
        mesh=mesh,
        compiler_params=pltpu.CompilerParams(use_tc_tiling_on_sc=False),
        out_type=jax.ShapeDtypeStruct((NUM_TOKENS, HIDDEN), jnp.float32),
        scratch_types=[
            pltpu.VMEM((N_CHUNKS, CHUNK), jnp.int32),
            pltpu.VMEM((ROWS_G, HIDDEN), jnp.float32),
            pltpu.VMEM((ROWS_G, HIDDEN), jnp.float32),
            pltpu.SemaphoreType.DMA,
            pltpu.SemaphoreType.DMA,
            pltpu.SemaphoreType.DMA,
            pltpu.SemaphoreType.DMA,
        ],
    )
    def k(idx_hbm, table_hbm, out_hbm, idx_v, buf_a, buf_b, gsem_a, gsem_b,
          osem_a, osem_b):
        wid = lax.axis_index("s") * 2 + lax.axis_index("c")
        base = wid * PER_WORKER
        pltpu.sync_copy(idx_hbm.at[wid], idx_v)

        def fire(g, buf, gsem):
            for j in range(G):
                pltpu.async_copy(
                    table_hbm.at[idx_v.at[g * G + j]],
                    buf.at[pl.ds(j * CHUNK, CHUNK)],
                    gsem,
                )

        def drain(buf, gsem):
            for j in range(G):
                pltpu.make_async_copy(
                    table_hbm.at[idx_v.at[j]],
                    buf.at[pl.ds(j * CHUNK, CHUNK)],
                    gsem,
                ).wait()

        def out_slice(g):
            return out_hbm.at[pl.ds(base + g * ROWS_G, ROWS_G)]

        def store(g, buf, osem):
            pltpu.async_copy(buf, out_slice(g), osem)

        def store_wait(g, buf, osem):
            pltpu.make_async_copy(buf, out_slice(g), osem).wait()

        fire(0, buf_a, gsem_a)
        fire(1, buf_b, gsem_b)
        drain(buf_a, gsem_a)
        store(0, buf_a, osem_a)

        def body(i, carry):
            store_wait(2 * i, buf_a, osem_a)
            fire(2 * i + 2, buf_a, gsem_a)
            drain(buf_b, gsem_b)
            store(2 * i + 1, buf_b, osem_b)
            store_wait(2 * i + 1, buf_b, osem_b)
            fire(2 * i + 3, buf_b, gsem_b)
            drain(buf_a, gsem_a)
            store(2 * i + 2, buf_a, osem_a)
            return carry

        lax.fori_loop(0, (NG - 2) // 2, body, 0)

        drain(buf_b, gsem_b)
        store(NG - 1, buf_b, osem_b)
        store_wait(NG - 2, buf_a, osem_a)
        store_wait(NG - 1, buf_b, osem_b)

    return k(idx, table)


def kernel(tokens, embedding_weight):
    b, t = tokens.shape
    idx = tokens.astype(jnp.int32).reshape(NUM_WORKERS, N_CHUNKS, CHUNK)
    out = _embed(idx, embedding_weight)
    return out.reshape(b, t, HIDDEN)
